# Initial kernel scaffold; baseline (speedup 1.0000x reference)
import functools

import jax
import jax.numpy as jnp
from jax import lax
from jax.experimental import pallas as pl
from jax.experimental.pallas import tpu as pltpu
from jax.experimental.pallas import tpu_sc as plsc

N = 10000
E = 320000
D = 128
L = 3

_NC = 2
_NS = 16
_CH = 80
_TILE_E = E // (_NC * _NS)
_ITERS = _TILE_E // _CH
_ZR = N // _NS

_ROWBLK = 1000


def _sc_agg(m, src2d, dst2d, zeros):
    mesh = plsc.VectorSubcoreMesh(core_axis_name="c", subcore_axis_name="s")

    @functools.partial(
        pl.kernel,
        out_type=jax.ShapeDtypeStruct((_NC, N, D), jnp.float32),
        mesh=mesh,
        scratch_types=[
            pltpu.VMEM((_ITERS, _CH), jnp.int32),
            pltpu.VMEM((_ITERS, _CH), jnp.int32),
            pltpu.VMEM((_CH, D), jnp.float32),
            pltpu.VMEM_SHARED((N, D), jnp.float32),
        ],
    )
    def k(m_hbm, src_hbm, dst_hbm, z_hbm, out_hbm, src_v, dst_v, rows_v, agg_sh):
        c = lax.axis_index("c")
        s = lax.axis_index("s")
        pltpu.sync_copy(z_hbm, agg_sh.at[pl.ds(s * _ZR, _ZR)])
        row0 = (c * _NS + s) * _ITERS
        pltpu.sync_copy(src_hbm.at[pl.ds(row0, _ITERS)], src_v)
        pltpu.sync_copy(dst_hbm.at[pl.ds(row0, _ITERS)], dst_v)
        plsc.subcore_barrier()

        @pl.loop(0, _ITERS)
        def _(i):
            pltpu.sync_copy(m_hbm.at[src_v.at[i]], rows_v)
            pltpu.sync_copy(rows_v, agg_sh.at[dst_v.at[i]], add=True)

        plsc.subcore_barrier()
        pltpu.sync_copy(agg_sh.at[pl.ds(s * _ZR, _ZR)],
                        out_hbm.at[c].at[pl.ds(s * _ZR, _ZR)])

    return k(m, src2d, dst2d, zeros)


def _tc_pre(h, wl, w_hh, b_hh):
    def body(h_ref, wl_ref, whh_ref, bhh_ref, m_ref, gh_ref):
        hb = h_ref[...]
        m_ref[...] = jnp.dot(hb, wl_ref[...], preferred_element_type=jnp.float32)
        gh_ref[...] = lax.dot_general(
            hb, whh_ref[...], (((1,), (1,)), ((), ())),
            preferred_element_type=jnp.float32) + bhh_ref[...]

    return pl.pallas_call(
        body,
        grid=(N // _ROWBLK,),
        in_specs=[
            pl.BlockSpec((_ROWBLK, D), lambda i: (i, 0)),
            pl.BlockSpec((D, D), lambda i: (0, 0)),
            pl.BlockSpec((3 * D, D), lambda i: (0, 0)),
            pl.BlockSpec((1, 3 * D), lambda i: (0, 0)),
        ],
        out_specs=[
            pl.BlockSpec((_ROWBLK, D), lambda i: (i, 0)),
            pl.BlockSpec((_ROWBLK, 3 * D), lambda i: (i, 0)),
        ],
        out_shape=[
            jax.ShapeDtypeStruct((N, D), jnp.float32),
            jax.ShapeDtypeStruct((N, 3 * D), jnp.float32),
        ],
    )(h, wl, w_hh, b_hh)


def _tc_gate(p, gh, h, w_ih, b_ih):
    def body(p_ref, gh_ref, h_ref, wih_ref, bih_ref, o_ref):
        agg = p_ref[0] + p_ref[1]
        gi = lax.dot_general(
            agg, wih_ref[...], (((1,), (1,)), ((), ())),
            preferred_element_type=jnp.float32) + bih_ref[...]
        ghb = gh_ref[...]
        hb = h_ref[...]
        r = jax.nn.sigmoid(gi[:, :D] + ghb[:, :D])
        z = jax.nn.sigmoid(gi[:, D:2 * D] + ghb[:, D:2 * D])
        n = jnp.tanh(gi[:, 2 * D:] + r * ghb[:, 2 * D:])
        o_ref[...] = (1.0 - z) * n + z * hb

    return pl.pallas_call(
        body,
        grid=(N // _ROWBLK,),
        in_specs=[
            pl.BlockSpec((_NC, _ROWBLK, D), lambda i: (0, i, 0)),
            pl.BlockSpec((_ROWBLK, 3 * D), lambda i: (i, 0)),
            pl.BlockSpec((_ROWBLK, D), lambda i: (i, 0)),
            pl.BlockSpec((3 * D, D), lambda i: (0, 0)),
            pl.BlockSpec((1, 3 * D), lambda i: (0, 0)),
        ],
        out_specs=pl.BlockSpec((_ROWBLK, D), lambda i: (i, 0)),
        out_shape=jax.ShapeDtypeStruct((N, D), jnp.float32),
    )(p, gh, h, w_ih, b_ih)


def _tc_fin(h, lin_w, lin_b):
    def body(h_ref, w_ref, b_ref, o_ref):
        hb = jnp.maximum(h_ref[...], 0.0)
        o_ref[...] = lax.dot_general(
            hb, w_ref[...], (((1,), (1,)), ((), ())),
            preferred_element_type=jnp.float32) + b_ref[...]

    return pl.pallas_call(
        body,
        grid=(N // _ROWBLK,),
        in_specs=[
            pl.BlockSpec((_ROWBLK, D), lambda i: (i, 0)),
            pl.BlockSpec((D, D), lambda i: (0, 0)),
            pl.BlockSpec((1, D), lambda i: (0, 0)),
        ],
        out_specs=pl.BlockSpec((_ROWBLK, D), lambda i: (i, 0)),
        out_shape=jax.ShapeDtypeStruct((N, D), jnp.float32),
    )(h, lin_b)


def kernel(x, edge_index, edge_attr, weight, W_ih, W_hh, b_ih, b_hh, emb, lin_W, lin_b):
    src2d = edge_index[0].reshape(E // _CH, _CH)
    dst2d = edge_index[1].reshape(E // _CH, _CH)
    zeros = jnp.zeros((_ZR, D), jnp.float32)
    b_ih2 = b_ih.reshape(1, 3 * D)
    b_hh2 = b_hh.reshape(1, 3 * D)
    lin_b2 = lin_b.reshape(1, D)

    h = x
    for l in range(L):
        m, gh = _tc_pre(h, weight[l], W_hh, b_hh2)
        p = _sc_agg(m, src2d, dst2d, zeros)
        h = _tc_gate(p, gh, h, W_ih, b_ih2)
    return _tc_fin(h, lin_W, lin_b2)


# speedup vs baseline: 7.5327x; 7.5327x over previous
import functools

import jax
import jax.numpy as jnp
from jax import lax
from jax.experimental import pallas as pl
from jax.experimental.pallas import tpu as pltpu
from jax.experimental.pallas import tpu_sc as plsc

N = 10000
E = 320000
D = 128
L = 3

_NC = 2
_NS = 16
_CH = 125
_TILE_E = E // (_NC * _NS)
_ITERS = _TILE_E // _CH
_ZR = 640
_NPAD = _ZR * _NS

_ROWBLK = 1000


def _sc_agg(m, src2d, dst2d, zeros):
    mesh = plsc.VectorSubcoreMesh(core_axis_name="c", subcore_axis_name="s")

    @functools.partial(
        pl.kernel,
        out_type=jax.ShapeDtypeStruct((_NC, N, D), jnp.float32),
        mesh=mesh,
        scratch_types=[
            pltpu.VMEM((_ITERS, _CH), jnp.int32),
            pltpu.VMEM((_ITERS, _CH), jnp.int32),
            pltpu.VMEM((_CH, D), jnp.float32),
            pltpu.VMEM_SHARED((_NPAD, D), jnp.float32),
        ],
    )
    def k(m_hbm, src_hbm, dst_hbm, z_hbm, out_hbm, src_v, dst_v, rows_v, agg_sh):
        c = lax.axis_index("c")
        s = lax.axis_index("s")
        pltpu.sync_copy(z_hbm, agg_sh.at[pl.ds(s * _ZR, _ZR)])
        row0 = (c * _NS + s) * _ITERS
        pltpu.sync_copy(src_hbm.at[pl.ds(row0, _ITERS)], src_v)
        pltpu.sync_copy(dst_hbm.at[pl.ds(row0, _ITERS)], dst_v)
        plsc.subcore_barrier()

        @pl.loop(0, _ITERS)
        def _(i):
            pltpu.sync_copy(m_hbm.at[src_v.at[i]], rows_v)
            pltpu.sync_copy(rows_v, agg_sh.at[dst_v.at[i]], add=True)

        plsc.subcore_barrier()
        ob = lax.min(s * _ZR, N - _ZR)
        pltpu.sync_copy(agg_sh.at[pl.ds(ob, _ZR)],
                        out_hbm.at[c].at[pl.ds(ob, _ZR)])

    return k(m, src2d, dst2d, zeros)


def _tc_pre(h, wl, w_hh, b_hh):
    def body(h_ref, wl_ref, whh_ref, bhh_ref, m_ref, gh_ref):
        hb = h_ref[...]
        m_ref[...] = jnp.dot(hb, wl_ref[...], preferred_element_type=jnp.float32)
        gh_ref[...] = lax.dot_general(
            hb, whh_ref[...], (((1,), (1,)), ((), ())),
            preferred_element_type=jnp.float32) + bhh_ref[...]

    return pl.pallas_call(
        body,
        grid=(N // _ROWBLK,),
        in_specs=[
            pl.BlockSpec((_ROWBLK, D), lambda i: (i, 0)),
            pl.BlockSpec((D, D), lambda i: (0, 0)),
            pl.BlockSpec((3 * D, D), lambda i: (0, 0)),
            pl.BlockSpec((1, 3 * D), lambda i: (0, 0)),
        ],
        out_specs=[
            pl.BlockSpec((_ROWBLK, D), lambda i: (i, 0)),
            pl.BlockSpec((_ROWBLK, 3 * D), lambda i: (i, 0)),
        ],
        out_shape=[
            jax.ShapeDtypeStruct((N, D), jnp.float32),
            jax.ShapeDtypeStruct((N, 3 * D), jnp.float32),
        ],
    )(h, wl, w_hh, b_hh)


def _tc_gate(p, gh, h, w_ih, b_ih):
    def body(p_ref, gh_ref, h_ref, wih_ref, bih_ref, o_ref):
        agg = p_ref[0] + p_ref[1]
        gi = lax.dot_general(
            agg, wih_ref[...], (((1,), (1,)), ((), ())),
            preferred_element_type=jnp.float32) + bih_ref[...]
        ghb = gh_ref[...]
        hb = h_ref[...]
        r = jax.nn.sigmoid(gi[:, :D] + ghb[:, :D])
        z = jax.nn.sigmoid(gi[:, D:2 * D] + ghb[:, D:2 * D])
        n = jnp.tanh(gi[:, 2 * D:] + r * ghb[:, 2 * D:])
        o_ref[...] = (1.0 - z) * n + z * hb

    return pl.pallas_call(
        body,
        grid=(N // _ROWBLK,),
        in_specs=[
            pl.BlockSpec((_NC, _ROWBLK, D), lambda i: (0, i, 0)),
            pl.BlockSpec((_ROWBLK, 3 * D), lambda i: (i, 0)),
            pl.BlockSpec((_ROWBLK, D), lambda i: (i, 0)),
            pl.BlockSpec((3 * D, D), lambda i: (0, 0)),
            pl.BlockSpec((1, 3 * D), lambda i: (0, 0)),
        ],
        out_specs=pl.BlockSpec((_ROWBLK, D), lambda i: (i, 0)),
        out_shape=jax.ShapeDtypeStruct((N, D), jnp.float32),
    )(p, gh, h, w_ih, b_ih)


def _tc_fin(h, lin_w, lin_b):
    def body(h_ref, w_ref, b_ref, o_ref):
        hb = jnp.maximum(h_ref[...], 0.0)
        o_ref[...] = lax.dot_general(
            hb, w_ref[...], (((1,), (1,)), ((), ())),
            preferred_element_type=jnp.float32) + b_ref[...]

    return pl.pallas_call(
        body,
        grid=(N // _ROWBLK,),
        in_specs=[
            pl.BlockSpec((_ROWBLK, D), lambda i: (i, 0)),
            pl.BlockSpec((D, D), lambda i: (0, 0)),
            pl.BlockSpec((1, D), lambda i: (0, 0)),
        ],
        out_specs=pl.BlockSpec((_ROWBLK, D), lambda i: (i, 0)),
        out_shape=jax.ShapeDtypeStruct((N, D), jnp.float32),
    )(h, lin_w, lin_b)


def kernel(x, edge_index, edge_attr, weight, W_ih, W_hh, b_ih, b_hh, emb, lin_W, lin_b):
    src2d = edge_index[0].reshape(E // _CH, _CH)
    dst2d = edge_index[1].reshape(E // _CH, _CH)
    zeros = jnp.zeros((_ZR, D), jnp.float32)
    b_ih2 = b_ih.reshape(1, 3 * D)
    b_hh2 = b_hh.reshape(1, 3 * D)
    lin_b2 = lin_b.reshape(1, D)

    h = x
    for l in range(L):
        m, gh = _tc_pre(h, weight[l], W_hh, b_hh2)
        p = _sc_agg(m, src2d, dst2d, zeros)
        h = _tc_gate(p, gh, h, W_ih, b_ih2)
    return _tc_fin(h, lin_W, lin_b2)


# speedup vs baseline: 10.9362x; 1.4518x over previous
import functools

import jax
import jax.numpy as jnp
from jax import lax
from jax.experimental import pallas as pl
from jax.experimental.pallas import tpu as pltpu
from jax.experimental.pallas import tpu_sc as plsc

N = 10000
E = 320000
D = 128
L = 3

_NC = 2
_NS = 16
_CH = 125
_TILE_E = E // (_NC * _NS)
_ITERS = _TILE_E // _CH
_ZR = 640
_NPAD = _ZR * _NS
_SB = 8
_NSB = _ITERS // _SB

_ROWBLK = 1000


def _sc_agg(m, src2d, dst2d, zeros):
    mesh = plsc.VectorSubcoreMesh(core_axis_name="c", subcore_axis_name="s")

    @functools.partial(
        pl.kernel,
        out_type=jax.ShapeDtypeStruct((_NC, N, D), jnp.float32),
        mesh=mesh,
        scratch_types=[
            pltpu.VMEM((_SB, _CH), jnp.int32),
            pltpu.VMEM((_SB, _CH), jnp.int32),
            pltpu.VMEM((_SB, _CH), jnp.int32),
            pltpu.VMEM((_SB, _CH), jnp.int32),
            pltpu.VMEM((_CH, D), jnp.float32),
            pltpu.VMEM((_CH, D), jnp.float32),
            pltpu.VMEM_SHARED((_NPAD, D), jnp.float32),
            pltpu.SemaphoreType.DMA,
            pltpu.SemaphoreType.DMA,
            pltpu.SemaphoreType.DMA,
            pltpu.SemaphoreType.DMA,
        ],
    )
    def k(m_hbm, src_hbm, dst_hbm, z_hbm, out_hbm,
          si0, si1, di0, di1, r0, r1, agg_sh, is0, is1, gs0, gs1):
        sidx = (si0, si1)
        didx = (di0, di1)
        data = (r0, r1)
        isem = (is0, is1)
        gsem = (gs0, gs1)
        c = lax.axis_index("c")
        s = lax.axis_index("s")
        pltpu.sync_copy(z_hbm, agg_sh.at[pl.ds(s * _ZR, _ZR)])
        q0 = (c * _NS + s) * _ITERS

        def idx_load(u, p):
            q = q0 + u * _SB
            pltpu.make_async_copy(src_hbm.at[pl.ds(q, _SB)], sidx[p], isem[p]).start()
            pltpu.make_async_copy(dst_hbm.at[pl.ds(q, _SB)], didx[p], isem[p]).start()

        def idx_wait(p):
            pltpu.make_async_copy(src_hbm.at[pl.ds(q0, _SB)], sidx[p], isem[p]).wait()
            pltpu.make_async_copy(dst_hbm.at[pl.ds(q0, _SB)], didx[p], isem[p]).wait()

        def gather_start(p, j, b):
            pltpu.make_async_copy(m_hbm.at[sidx[p].at[j]], data[b], gsem[b]).start()

        def gather_wait(p, j, b):
            pltpu.make_async_copy(m_hbm.at[sidx[p].at[j]], data[b], gsem[b]).wait()

        def super_visit(u, p, load_next2, next_block):
            for j in range(_SB):
                b = j % 2
                if j < _SB - 1:
                    gather_start(p, j + 1, (j + 1) % 2)
                elif next_block:
                    idx_wait(1 - p)
                    gather_start(1 - p, 0, 0)
                gather_wait(p, j, b)
                pltpu.sync_copy(data[b], agg_sh.at[didx[p].at[j]], add=True)
                if j == _SB - 1 and load_next2:
                    idx_load(u + 2, p)

        plsc.subcore_barrier()

        idx_load(0, 0)
        idx_load(1, 1)
        idx_wait(0)
        gather_start(0, 0, 0)

        @pl.loop(0, (_NSB - 2) // 2)
        def _(i):
            super_visit(2 * i, 0, True, True)
            super_visit(2 * i + 1, 1, True, True)

        super_visit(_NSB - 2, 0, False, True)
        super_visit(_NSB - 1, 1, False, False)

        plsc.subcore_barrier()
        ob = lax.min(s * _ZR, N - _ZR)
        pltpu.sync_copy(agg_sh.at[pl.ds(ob, _ZR)],
                        out_hbm.at[c].at[pl.ds(ob, _ZR)])

    return k(m, src2d, dst2d, zeros)


def _tc_pre(h, wl, w_hh, b_hh):
    def body(h_ref, wl_ref, whh_ref, bhh_ref, m_ref, gh_ref):
        hb = h_ref[...]
        m_ref[...] = jnp.dot(hb, wl_ref[...], preferred_element_type=jnp.float32)
        gh_ref[...] = lax.dot_general(
            hb, whh_ref[...], (((1,), (1,)), ((), ())),
            preferred_element_type=jnp.float32) + bhh_ref[...]

    return pl.pallas_call(
        body,
        grid=(N // _ROWBLK,),
        in_specs=[
            pl.BlockSpec((_ROWBLK, D), lambda i: (i, 0)),
            pl.BlockSpec((D, D), lambda i: (0, 0)),
            pl.BlockSpec((3 * D, D), lambda i: (0, 0)),
            pl.BlockSpec((1, 3 * D), lambda i: (0, 0)),
        ],
        out_specs=[
            pl.BlockSpec((_ROWBLK, D), lambda i: (i, 0)),
            pl.BlockSpec((_ROWBLK, 3 * D), lambda i: (i, 0)),
        ],
        out_shape=[
            jax.ShapeDtypeStruct((N, D), jnp.float32),
            jax.ShapeDtypeStruct((N, 3 * D), jnp.float32),
        ],
    )(h, wl, w_hh, b_hh)


def _tc_gate(p, gh, h, w_ih, b_ih):
    def body(p_ref, gh_ref, h_ref, wih_ref, bih_ref, o_ref):
        agg = p_ref[0] + p_ref[1]
        gi = lax.dot_general(
            agg, wih_ref[...], (((1,), (1,)), ((), ())),
            preferred_element_type=jnp.float32) + bih_ref[...]
        ghb = gh_ref[...]
        hb = h_ref[...]
        r = jax.nn.sigmoid(gi[:, :D] + ghb[:, :D])
        z = jax.nn.sigmoid(gi[:, D:2 * D] + ghb[:, D:2 * D])
        n = jnp.tanh(gi[:, 2 * D:] + r * ghb[:, 2 * D:])
        o_ref[...] = (1.0 - z) * n + z * hb

    return pl.pallas_call(
        body,
        grid=(N // _ROWBLK,),
        in_specs=[
            pl.BlockSpec((_NC, _ROWBLK, D), lambda i: (0, i, 0)),
            pl.BlockSpec((_ROWBLK, 3 * D), lambda i: (i, 0)),
            pl.BlockSpec((_ROWBLK, D), lambda i: (i, 0)),
            pl.BlockSpec((3 * D, D), lambda i: (0, 0)),
            pl.BlockSpec((1, 3 * D), lambda i: (0, 0)),
        ],
        out_specs=pl.BlockSpec((_ROWBLK, D), lambda i: (i, 0)),
        out_shape=jax.ShapeDtypeStruct((N, D), jnp.float32),
    )(p, gh, h, w_ih, b_ih)


def _tc_fin(h, lin_w, lin_b):
    def body(h_ref, w_ref, b_ref, o_ref):
        hb = jnp.maximum(h_ref[...], 0.0)
        o_ref[...] = lax.dot_general(
            hb, w_ref[...], (((1,), (1,)), ((), ())),
            preferred_element_type=jnp.float32) + b_ref[...]

    return pl.pallas_call(
        body,
        grid=(N // _ROWBLK,),
        in_specs=[
            pl.BlockSpec((_ROWBLK, D), lambda i: (i, 0)),
            pl.BlockSpec((D, D), lambda i: (0, 0)),
            pl.BlockSpec((1, D), lambda i: (0, 0)),
        ],
        out_specs=pl.BlockSpec((_ROWBLK, D), lambda i: (i, 0)),
        out_shape=jax.ShapeDtypeStruct((N, D), jnp.float32),
    )(h, lin_w, lin_b)


def kernel(x, edge_index, edge_attr, weight, W_ih, W_hh, b_ih, b_hh, emb, lin_W, lin_b):
    src2d = edge_index[0].reshape(E // _CH, _CH)
    dst2d = edge_index[1].reshape(E // _CH, _CH)
    zeros = jnp.zeros((_ZR, D), jnp.float32)
    b_ih2 = b_ih.reshape(1, 3 * D)
    b_hh2 = b_hh.reshape(1, 3 * D)
    lin_b2 = lin_b.reshape(1, D)

    h = x
    for l in range(L):
        m, gh = _tc_pre(h, weight[l], W_hh, b_hh2)
        p = _sc_agg(m, src2d, dst2d, zeros)
        h = _tc_gate(p, gh, h, W_ih, b_ih2)
    return _tc_fin(h, lin_W, lin_b2)


# speedup vs baseline: 11.8995x; 1.0881x over previous
import functools

import jax
import jax.numpy as jnp
from jax import lax
from jax.experimental import pallas as pl
from jax.experimental.pallas import tpu as pltpu
from jax.experimental.pallas import tpu_sc as plsc

N = 10000
E = 320000
D = 128
L = 3

_NC = 2
_NS = 16
_CH = 125
_TILE_E = E // (_NC * _NS)
_ITERS = _TILE_E // _CH
_ZR = 640
_NPAD = _ZR * _NS
_SB = 8
_NSB = _ITERS // _SB

_ROWBLK = 1000


def _sc_agg(m, src2d, dst2d, zeros):
    mesh = plsc.VectorSubcoreMesh(core_axis_name="c", subcore_axis_name="s")

    @functools.partial(
        pl.kernel,
        out_type=jax.ShapeDtypeStruct((_NC, N, D), jnp.float32),
        mesh=mesh,
        scratch_types=[
            pltpu.VMEM((_SB, _CH), jnp.int32),
            pltpu.VMEM((_SB, _CH), jnp.int32),
            pltpu.VMEM((_SB, _CH), jnp.int32),
            pltpu.VMEM((_SB, _CH), jnp.int32),
            pltpu.VMEM((_CH, D), jnp.float32),
            pltpu.VMEM((_CH, D), jnp.float32),
            pltpu.VMEM_SHARED((_NPAD, D), jnp.float32),
            pltpu.SemaphoreType.DMA,
            pltpu.SemaphoreType.DMA,
            pltpu.SemaphoreType.DMA,
            pltpu.SemaphoreType.DMA,
            pltpu.SemaphoreType.DMA,
            pltpu.SemaphoreType.DMA,
        ],
    )
    def k(m_hbm, src_hbm, dst_hbm, z_hbm, out_hbm,
          si0, si1, di0, di1, r0, r1, agg_sh, is0, is1, gs0, gs1, ss0, ss1):
        sidx = (si0, si1)
        didx = (di0, di1)
        data = (r0, r1)
        isem = (is0, is1)
        gsem = (gs0, gs1)
        ssem = (ss0, ss1)
        c = lax.axis_index("c")
        s = lax.axis_index("s")
        pltpu.sync_copy(z_hbm, agg_sh.at[pl.ds(s * _ZR, _ZR)])
        q0 = (c * _NS + s) * _ITERS

        def idx_load(u, p):
            q = q0 + u * _SB
            pltpu.make_async_copy(src_hbm.at[pl.ds(q, _SB)], sidx[p], isem[p]).start()
            pltpu.make_async_copy(dst_hbm.at[pl.ds(q, _SB)], didx[p], isem[p]).start()

        def idx_wait(p):
            pltpu.make_async_copy(src_hbm.at[pl.ds(q0, _SB)], sidx[p], isem[p]).wait()
            pltpu.make_async_copy(dst_hbm.at[pl.ds(q0, _SB)], didx[p], isem[p]).wait()

        def gather_start(p, j, b):
            pltpu.make_async_copy(m_hbm.at[sidx[p].at[j]], data[b], gsem[b]).start()

        def gather_wait(p, j, b):
            pltpu.make_async_copy(m_hbm.at[sidx[p].at[j]], data[b], gsem[b]).wait()

        def scatter_start(p, j, b):
            pltpu.async_copy(data[b], agg_sh.at[didx[p].at[j]], ssem[b], add=True)

        def scatter_wait(p, j, b):
            pltpu.make_async_copy(data[b], agg_sh.at[didx[p].at[j]], ssem[b]).wait()

        def super_visit(u, p, load_next2, next_block, first=False):
            for j in range(_SB):
                b = j % 2
                nb = (j + 1) % 2
                if j < _SB - 1:
                    if not (first and j == 0):
                        scatter_wait(p, j, nb)
                    gather_start(p, j + 1, nb)
                elif next_block:
                    scatter_wait(p, j, nb)
                    idx_wait(1 - p)
                    gather_start(1 - p, 0, 0)
                gather_wait(p, j, b)
                scatter_start(p, j, b)
                if j == _SB - 1 and load_next2:
                    idx_load(u + 2, p)

        plsc.subcore_barrier()

        idx_load(0, 0)
        idx_load(1, 1)
        idx_wait(0)
        gather_start(0, 0, 0)

        super_visit(0, 0, True, True, first=True)
        super_visit(1, 1, True, True)

        @pl.loop(0, (_NSB - 4) // 2)
        def _(i):
            super_visit(2 * i + 2, 0, True, True)
            super_visit(2 * i + 3, 1, True, True)

        super_visit(_NSB - 2, 0, False, True)
        super_visit(_NSB - 1, 1, False, False)
        scatter_wait(1, _SB - 2, 0)
        scatter_wait(1, _SB - 1, 1)

        plsc.subcore_barrier()
        ob = lax.min(s * _ZR, N - _ZR)
        pltpu.sync_copy(agg_sh.at[pl.ds(ob, _ZR)],
                        out_hbm.at[c].at[pl.ds(ob, _ZR)])

    return k(m, src2d, dst2d, zeros)


def _tc_m(h, wl):
    def body(h_ref, wl_ref, m_ref):
        m_ref[...] = jnp.dot(h_ref[...], wl_ref[...],
                             preferred_element_type=jnp.float32)

    return pl.pallas_call(
        body,
        grid=(N // _ROWBLK,),
        in_specs=[
            pl.BlockSpec((_ROWBLK, D), lambda i: (i, 0)),
            pl.BlockSpec((D, D), lambda i: (0, 0)),
        ],
        out_specs=pl.BlockSpec((_ROWBLK, D), lambda i: (i, 0)),
        out_shape=jax.ShapeDtypeStruct((N, D), jnp.float32),
    )(h, wl)


def _gru(p_ref, h_ref, wih_ref, bih_ref, whh_ref, bhh_ref):
    agg = p_ref[0] + p_ref[1]
    gi = lax.dot_general(
        agg, wih_ref[...], (((1,), (1,)), ((), ())),
        preferred_element_type=jnp.float32) + bih_ref[...]
    gh = lax.dot_general(
        h_ref[...], whh_ref[...], (((1,), (1,)), ((), ())),
        preferred_element_type=jnp.float32) + bhh_ref[...]
    r = jax.nn.sigmoid(gi[:, :D] + gh[:, :D])
    z = jax.nn.sigmoid(gi[:, D:2 * D] + gh[:, D:2 * D])
    n = jnp.tanh(gi[:, 2 * D:] + r * gh[:, 2 * D:])
    return (1.0 - z) * n + z * h_ref[...]


def _tc_gate(p, h, w_ih, b_ih, w_hh, b_hh, wnext):
    def body(p_ref, h_ref, wih_ref, bih_ref, whh_ref, bhh_ref, wn_ref,
             hn_ref, m_ref):
        hn = _gru(p_ref, h_ref, wih_ref, bih_ref, whh_ref, bhh_ref)
        hn_ref[...] = hn
        m_ref[...] = jnp.dot(hn, wn_ref[...], preferred_element_type=jnp.float32)

    return pl.pallas_call(
        body,
        grid=(N // _ROWBLK,),
        in_specs=[
            pl.BlockSpec((_NC, _ROWBLK, D), lambda i: (0, i, 0)),
            pl.BlockSpec((_ROWBLK, D), lambda i: (i, 0)),
            pl.BlockSpec((3 * D, D), lambda i: (0, 0)),
            pl.BlockSpec((1, 3 * D), lambda i: (0, 0)),
            pl.BlockSpec((3 * D, D), lambda i: (0, 0)),
            pl.BlockSpec((1, 3 * D), lambda i: (0, 0)),
            pl.BlockSpec((D, D), lambda i: (0, 0)),
        ],
        out_specs=[
            pl.BlockSpec((_ROWBLK, D), lambda i: (i, 0)),
            pl.BlockSpec((_ROWBLK, D), lambda i: (i, 0)),
        ],
        out_shape=[
            jax.ShapeDtypeStruct((N, D), jnp.float32),
            jax.ShapeDtypeStruct((N, D), jnp.float32),
        ],
    )(p, h, w_ih, b_ih, w_hh, b_hh, wnext)


def _tc_gate_fin(p, h, w_ih, b_ih, w_hh, b_hh, lin_w, lin_b):
    def body(p_ref, h_ref, wih_ref, bih_ref, whh_ref, bhh_ref, w_ref, b_ref,
             o_ref):
        hn = _gru(p_ref, h_ref, wih_ref, bih_ref, whh_ref, bhh_ref)
        o_ref[...] = lax.dot_general(
            jnp.maximum(hn, 0.0), w_ref[...], (((1,), (1,)), ((), ())),
            preferred_element_type=jnp.float32) + b_ref[...]

    return pl.pallas_call(
        body,
        grid=(N // _ROWBLK,),
        in_specs=[
            pl.BlockSpec((_NC, _ROWBLK, D), lambda i: (0, i, 0)),
            pl.BlockSpec((_ROWBLK, D), lambda i: (i, 0)),
            pl.BlockSpec((3 * D, D), lambda i: (0, 0)),
            pl.BlockSpec((1, 3 * D), lambda i: (0, 0)),
            pl.BlockSpec((3 * D, D), lambda i: (0, 0)),
            pl.BlockSpec((1, 3 * D), lambda i: (0, 0)),
            pl.BlockSpec((D, D), lambda i: (0, 0)),
            pl.BlockSpec((1, D), lambda i: (0, 0)),
        ],
        out_specs=pl.BlockSpec((_ROWBLK, D), lambda i: (i, 0)),
        out_shape=jax.ShapeDtypeStruct((N, D), jnp.float32),
    )(p, h, w_ih, b_ih, w_hh, b_hh, lin_w, lin_b)


def kernel(x, edge_index, edge_attr, weight, W_ih, W_hh, b_ih, b_hh, emb, lin_W, lin_b):
    src2d = edge_index[0].reshape(E // _CH, _CH)
    dst2d = edge_index[1].reshape(E // _CH, _CH)
    zeros = jnp.zeros((_ZR, D), jnp.float32)
    b_ih2 = b_ih.reshape(1, 3 * D)
    b_hh2 = b_hh.reshape(1, 3 * D)
    lin_b2 = lin_b.reshape(1, D)

    h = x
    m = _tc_m(h, weight[0])
    for l in range(L - 1):
        p = _sc_agg(m, src2d, dst2d, zeros)
        h, m = _tc_gate(p, h, W_ih, b_ih2, W_hh, b_hh2, weight[l + 1])
    p = _sc_agg(m, src2d, dst2d, zeros)
    return _tc_gate_fin(p, h, W_ih, b_ih2, W_hh, b_hh2, lin_W, lin_b2)


# speedup vs baseline: 12.0904x; 1.0160x over previous
import functools

import jax
import jax.numpy as jnp
from jax import lax
from jax.experimental import pallas as pl
from jax.experimental.pallas import tpu as pltpu
from jax.experimental.pallas import tpu_sc as plsc

N = 10000
E = 320000
D = 128
L = 3

_NC = 2
_NS = 16
_CH = 125
_TILE_E = E // (_NC * _NS)
_ITERS = _TILE_E // _CH
_ZR = 640
_NPAD = _ZR * _NS
_SB = 8
_NSB = _ITERS // _SB

_ROWBLK = 1000


def _sc_agg(m, src2d, dst2d, zeros):
    mesh = plsc.VectorSubcoreMesh(core_axis_name="c", subcore_axis_name="s")

    @functools.partial(
        pl.kernel,
        out_type=jax.ShapeDtypeStruct((_NC, N, D), jnp.float32),
        mesh=mesh,
        scratch_types=[
            pltpu.VMEM((_SB, _CH), jnp.int32),
            pltpu.VMEM((_SB, _CH), jnp.int32),
            pltpu.VMEM((_SB, _CH), jnp.int32),
            pltpu.VMEM((_SB, _CH), jnp.int32),
            pltpu.VMEM((_CH, D), jnp.float32),
            pltpu.VMEM((_CH, D), jnp.float32),
            pltpu.VMEM_SHARED((_NPAD, D), jnp.float32),
            pltpu.SemaphoreType.DMA,
            pltpu.SemaphoreType.DMA,
            pltpu.SemaphoreType.DMA,
            pltpu.SemaphoreType.DMA,
            pltpu.SemaphoreType.DMA,
            pltpu.SemaphoreType.DMA,
            pltpu.SemaphoreType.DMA,
        ],
    )
    def k(m_hbm, src_hbm, dst_hbm, z_hbm, out_hbm,
          si0, si1, di0, di1, r0, r1, agg_sh, is0, is1, gs0, gs1, ss0, ss1,
          zsem):
        sidx = (si0, si1)
        didx = (di0, di1)
        data = (r0, r1)
        isem = (is0, is1)
        gsem = (gs0, gs1)
        ssem = (ss0, ss1)
        c = lax.axis_index("c")
        s = lax.axis_index("s")
        zcp = pltpu.make_async_copy(z_hbm, agg_sh.at[pl.ds(s * _ZR, _ZR)], zsem)
        zcp.start()
        q0 = (c * _NS + s) * _ITERS

        def idx_load(u, p):
            q = q0 + u * _SB
            pltpu.make_async_copy(src_hbm.at[pl.ds(q, _SB)], sidx[p], isem[p]).start()
            pltpu.make_async_copy(dst_hbm.at[pl.ds(q, _SB)], didx[p], isem[p]).start()

        def idx_wait(p):
            pltpu.make_async_copy(src_hbm.at[pl.ds(q0, _SB)], sidx[p], isem[p]).wait()
            pltpu.make_async_copy(dst_hbm.at[pl.ds(q0, _SB)], didx[p], isem[p]).wait()

        def gather_start(p, j, b):
            pltpu.make_async_copy(m_hbm.at[sidx[p].at[j]], data[b], gsem[b]).start()

        def gather_wait(p, j, b):
            pltpu.make_async_copy(m_hbm.at[sidx[p].at[j]], data[b], gsem[b]).wait()

        def scatter_start(p, j, b):
            pltpu.async_copy(data[b], agg_sh.at[didx[p].at[j]], ssem[b], add=True)

        def scatter_wait(p, j, b):
            pltpu.make_async_copy(data[b], agg_sh.at[didx[p].at[j]], ssem[b]).wait()

        def super_visit(u, p, load_next2, next_block, first=False):
            for j in range(_SB):
                b = j % 2
                nb = (j + 1) % 2
                if j < _SB - 1:
                    if not (first and j == 0):
                        scatter_wait(p, j, nb)
                    gather_start(p, j + 1, nb)
                elif next_block:
                    scatter_wait(p, j, nb)
                    idx_wait(1 - p)
                    gather_start(1 - p, 0, 0)
                gather_wait(p, j, b)
                scatter_start(p, j, b)
                if j == _SB - 1 and load_next2:
                    idx_load(u + 2, p)

        idx_load(0, 0)
        idx_load(1, 1)
        idx_wait(0)
        gather_start(0, 0, 0)
        zcp.wait()
        plsc.subcore_barrier()

        super_visit(0, 0, True, True, first=True)
        super_visit(1, 1, True, True)

        @pl.loop(0, (_NSB - 4) // 2)
        def _(i):
            super_visit(2 * i + 2, 0, True, True)
            super_visit(2 * i + 3, 1, True, True)

        super_visit(_NSB - 2, 0, False, True)
        super_visit(_NSB - 1, 1, False, False)
        scatter_wait(1, _SB - 2, 0)
        scatter_wait(1, _SB - 1, 1)

        plsc.subcore_barrier()
        ob = lax.min(s * _ZR, N - _ZR)
        pltpu.sync_copy(agg_sh.at[pl.ds(ob, _ZR)],
                        out_hbm.at[c].at[pl.ds(ob, _ZR)])

    return k(m, src2d, dst2d, zeros)


def _tc_m(h, wl):
    def body(h_ref, wl_ref, m_ref):
        m_ref[...] = jnp.dot(h_ref[...].astype(jnp.bfloat16), wl_ref[...],
                             preferred_element_type=jnp.float32)

    return pl.pallas_call(
        body,
        grid=(N // _ROWBLK,),
        in_specs=[
            pl.BlockSpec((_ROWBLK, D), lambda i: (i, 0)),
            pl.BlockSpec((D, D), lambda i: (0, 0)),
        ],
        out_specs=pl.BlockSpec((_ROWBLK, D), lambda i: (i, 0)),
        out_shape=jax.ShapeDtypeStruct((N, D), jnp.float32),
    )(h, wl)


def _gru(p_ref, h_ref, wih_ref, bih_ref, whh_ref, bhh_ref):
    agg = (p_ref[0] + p_ref[1]).astype(jnp.bfloat16)
    gi = lax.dot_general(
        agg, wih_ref[...], (((1,), (1,)), ((), ())),
        preferred_element_type=jnp.float32) + bih_ref[...]
    gh = lax.dot_general(
        h_ref[...].astype(jnp.bfloat16), whh_ref[...], (((1,), (1,)), ((), ())),
        preferred_element_type=jnp.float32) + bhh_ref[...]
    r = jax.nn.sigmoid(gi[:, :D] + gh[:, :D])
    z = jax.nn.sigmoid(gi[:, D:2 * D] + gh[:, D:2 * D])
    n = jnp.tanh(gi[:, 2 * D:] + r * gh[:, 2 * D:])
    return (1.0 - z) * n + z * h_ref[...]


def _tc_gate(p, h, w_ih, b_ih, w_hh, b_hh, wnext):
    def body(p_ref, h_ref, wih_ref, bih_ref, whh_ref, bhh_ref, wn_ref,
             hn_ref, m_ref):
        hn = _gru(p_ref, h_ref, wih_ref, bih_ref, whh_ref, bhh_ref)
        hn_ref[...] = hn
        m_ref[...] = jnp.dot(hn.astype(jnp.bfloat16), wn_ref[...],
                             preferred_element_type=jnp.float32)

    return pl.pallas_call(
        body,
        grid=(N // _ROWBLK,),
        in_specs=[
            pl.BlockSpec((_NC, _ROWBLK, D), lambda i: (0, i, 0)),
            pl.BlockSpec((_ROWBLK, D), lambda i: (i, 0)),
            pl.BlockSpec((3 * D, D), lambda i: (0, 0)),
            pl.BlockSpec((1, 3 * D), lambda i: (0, 0)),
            pl.BlockSpec((3 * D, D), lambda i: (0, 0)),
            pl.BlockSpec((1, 3 * D), lambda i: (0, 0)),
            pl.BlockSpec((D, D), lambda i: (0, 0)),
        ],
        out_specs=[
            pl.BlockSpec((_ROWBLK, D), lambda i: (i, 0)),
            pl.BlockSpec((_ROWBLK, D), lambda i: (i, 0)),
        ],
        out_shape=[
            jax.ShapeDtypeStruct((N, D), jnp.float32),
            jax.ShapeDtypeStruct((N, D), jnp.float32),
        ],
    )(p, h, w_ih, b_ih, w_hh, b_hh, wnext)


def _tc_gate_fin(p, h, w_ih, b_ih, w_hh, b_hh, lin_w, lin_b):
    def body(p_ref, h_ref, wih_ref, bih_ref, whh_ref, bhh_ref, w_ref, b_ref,
             o_ref):
        hn = _gru(p_ref, h_ref, wih_ref, bih_ref, whh_ref, bhh_ref)
        o_ref[...] = lax.dot_general(
            jnp.maximum(hn, 0.0).astype(jnp.bfloat16), w_ref[...],
            (((1,), (1,)), ((), ())),
            preferred_element_type=jnp.float32) + b_ref[...]

    return pl.pallas_call(
        body,
        grid=(N // _ROWBLK,),
        in_specs=[
            pl.BlockSpec((_NC, _ROWBLK, D), lambda i: (0, i, 0)),
            pl.BlockSpec((_ROWBLK, D), lambda i: (i, 0)),
            pl.BlockSpec((3 * D, D), lambda i: (0, 0)),
            pl.BlockSpec((1, 3 * D), lambda i: (0, 0)),
            pl.BlockSpec((3 * D, D), lambda i: (0, 0)),
            pl.BlockSpec((1, 3 * D), lambda i: (0, 0)),
            pl.BlockSpec((D, D), lambda i: (0, 0)),
            pl.BlockSpec((1, D), lambda i: (0, 0)),
        ],
        out_specs=pl.BlockSpec((_ROWBLK, D), lambda i: (i, 0)),
        out_shape=jax.ShapeDtypeStruct((N, D), jnp.float32),
    )(p, h, w_ih, b_ih, w_hh, b_hh, lin_w, lin_b)


def kernel(x, edge_index, edge_attr, weight, W_ih, W_hh, b_ih, b_hh, emb, lin_W, lin_b):
    src2d = edge_index[0].reshape(E // _CH, _CH)
    dst2d = edge_index[1].reshape(E // _CH, _CH)
    zeros = jnp.zeros((_ZR, D), jnp.float32)
    b_ih2 = b_ih.reshape(1, 3 * D)
    b_hh2 = b_hh.reshape(1, 3 * D)
    lin_b2 = lin_b.reshape(1, D)
    wb = weight.astype(jnp.bfloat16)
    wih_b = W_ih.astype(jnp.bfloat16)
    whh_b = W_hh.astype(jnp.bfloat16)
    lin_wb = lin_W.astype(jnp.bfloat16)

    h = x
    m = _tc_m(h, wb[0])
    for l in range(L - 1):
        p = _sc_agg(m, src2d, dst2d, zeros)
        h, m = _tc_gate(p, h, wih_b, b_ih2, whh_b, b_hh2, wb[l + 1])
    p = _sc_agg(m, src2d, dst2d, zeros)
    return _tc_gate_fin(p, h, wih_b, b_ih2, whh_b, b_hh2, lin_wb, lin_b2)


# speedup vs baseline: 12.3849x; 1.0244x over previous
import functools

import jax
import jax.numpy as jnp
from jax import lax
from jax.experimental import pallas as pl
from jax.experimental.pallas import tpu as pltpu
from jax.experimental.pallas import tpu_sc as plsc

N = 10000
E = 320000
D = 128
L = 3

_NC = 2
_NS = 16
_CH = 125
_TILE_E = E // (_NC * _NS)
_ITERS = _TILE_E // _CH
_ZR = 640
_NPAD = _ZR * _NS
_SB = 8
_NSB = _ITERS // _SB

_ROWBLK = 2000


def _sc_agg(m, src2d, dst2d, zeros):
    mesh = plsc.VectorSubcoreMesh(core_axis_name="c", subcore_axis_name="s")

    @functools.partial(
        pl.kernel,
        out_type=jax.ShapeDtypeStruct((_NC, N, D), jnp.float32),
        mesh=mesh,
        scratch_types=[
            pltpu.VMEM((_SB, _CH), jnp.int32),
            pltpu.VMEM((_SB, _CH), jnp.int32),
            pltpu.VMEM((_SB, _CH), jnp.int32),
            pltpu.VMEM((_SB, _CH), jnp.int32),
            pltpu.VMEM((_CH, D), jnp.float32),
            pltpu.VMEM((_CH, D), jnp.float32),
            pltpu.VMEM_SHARED((_NPAD, D), jnp.float32),
            pltpu.SemaphoreType.DMA,
            pltpu.SemaphoreType.DMA,
            pltpu.SemaphoreType.DMA,
            pltpu.SemaphoreType.DMA,
            pltpu.SemaphoreType.DMA,
            pltpu.SemaphoreType.DMA,
            pltpu.SemaphoreType.DMA,
        ],
    )
    def k(m_hbm, src_hbm, dst_hbm, z_hbm, out_hbm,
          si0, si1, di0, di1, r0, r1, agg_sh, is0, is1, gs0, gs1, ss0, ss1,
          zsem):
        sidx = (si0, si1)
        didx = (di0, di1)
        data = (r0, r1)
        isem = (is0, is1)
        gsem = (gs0, gs1)
        ssem = (ss0, ss1)
        c = lax.axis_index("c")
        s = lax.axis_index("s")
        zcp = pltpu.make_async_copy(z_hbm, agg_sh.at[pl.ds(s * _ZR, _ZR)], zsem)
        zcp.start()
        q0 = (c * _NS + s) * _ITERS

        def idx_load(u, p):
            q = q0 + u * _SB
            pltpu.make_async_copy(src_hbm.at[pl.ds(q, _SB)], sidx[p], isem[p]).start()
            pltpu.make_async_copy(dst_hbm.at[pl.ds(q, _SB)], didx[p], isem[p]).start()

        def idx_wait(p):
            pltpu.make_async_copy(src_hbm.at[pl.ds(q0, _SB)], sidx[p], isem[p]).wait()
            pltpu.make_async_copy(dst_hbm.at[pl.ds(q0, _SB)], didx[p], isem[p]).wait()

        def gather_start(p, j, b):
            pltpu.make_async_copy(m_hbm.at[sidx[p].at[j]], data[b], gsem[b]).start()

        def gather_wait(p, j, b):
            pltpu.make_async_copy(m_hbm.at[sidx[p].at[j]], data[b], gsem[b]).wait()

        def scatter_start(p, j, b):
            pltpu.async_copy(data[b], agg_sh.at[didx[p].at[j]], ssem[b], add=True)

        def scatter_wait(p, j, b):
            pltpu.make_async_copy(data[b], agg_sh.at[didx[p].at[j]], ssem[b]).wait()

        def super_visit(u, p, load_next2, next_block, first=False):
            for j in range(_SB):
                b = j % 2
                nb = (j + 1) % 2
                if j < _SB - 1:
                    if not (first and j == 0):
                        scatter_wait(p, j, nb)
                    gather_start(p, j + 1, nb)
                elif next_block:
                    scatter_wait(p, j, nb)
                    idx_wait(1 - p)
                    gather_start(1 - p, 0, 0)
                gather_wait(p, j, b)
                scatter_start(p, j, b)
                if j == _SB - 1 and load_next2:
                    idx_load(u + 2, p)

        idx_load(0, 0)
        idx_load(1, 1)
        idx_wait(0)
        gather_start(0, 0, 0)
        zcp.wait()
        plsc.subcore_barrier()

        super_visit(0, 0, True, True, first=True)
        super_visit(1, 1, True, True)

        @pl.loop(0, (_NSB - 4) // 2)
        def _(i):
            super_visit(2 * i + 2, 0, True, True)
            super_visit(2 * i + 3, 1, True, True)

        super_visit(_NSB - 2, 0, False, True)
        super_visit(_NSB - 1, 1, False, False)
        scatter_wait(1, _SB - 2, 0)
        scatter_wait(1, _SB - 1, 1)

        plsc.subcore_barrier()
        ob = lax.min(s * _ZR, N - _ZR)
        pltpu.sync_copy(agg_sh.at[pl.ds(ob, _ZR)],
                        out_hbm.at[c].at[pl.ds(ob, _ZR)])

    return k(m, src2d, dst2d, zeros)


def _tc_m(h, wl):
    def body(h_ref, wl_ref, m_ref):
        m_ref[...] = jnp.dot(h_ref[...].astype(jnp.bfloat16), wl_ref[...],
                             preferred_element_type=jnp.float32)

    return pl.pallas_call(
        body,
        grid=(N // _ROWBLK,),
        in_specs=[
            pl.BlockSpec((_ROWBLK, D), lambda i: (i, 0)),
            pl.BlockSpec((D, D), lambda i: (0, 0)),
        ],
        out_specs=pl.BlockSpec((_ROWBLK, D), lambda i: (i, 0)),
        out_shape=jax.ShapeDtypeStruct((N, D), jnp.float32),
    )(h, wl)


def _gru(p_ref, h_ref, wih_ref, bih_ref, whh_ref, bhh_ref):
    agg = (p_ref[0] + p_ref[1]).astype(jnp.bfloat16)
    gi = lax.dot_general(
        agg, wih_ref[...], (((1,), (1,)), ((), ())),
        preferred_element_type=jnp.float32) + bih_ref[...]
    gh = lax.dot_general(
        h_ref[...].astype(jnp.bfloat16), whh_ref[...], (((1,), (1,)), ((), ())),
        preferred_element_type=jnp.float32) + bhh_ref[...]
    r = jax.nn.sigmoid(gi[:, :D] + gh[:, :D])
    z = jax.nn.sigmoid(gi[:, D:2 * D] + gh[:, D:2 * D])
    n = jnp.tanh(gi[:, 2 * D:] + r * gh[:, 2 * D:])
    return (1.0 - z) * n + z * h_ref[...]


def _tc_gate(p, h, w_ih, b_ih, w_hh, b_hh, wnext):
    def body(p_ref, h_ref, wih_ref, bih_ref, whh_ref, bhh_ref, wn_ref,
             hn_ref, m_ref):
        hn = _gru(p_ref, h_ref, wih_ref, bih_ref, whh_ref, bhh_ref)
        hn_ref[...] = hn
        m_ref[...] = jnp.dot(hn.astype(jnp.bfloat16), wn_ref[...],
                             preferred_element_type=jnp.float32)

    return pl.pallas_call(
        body,
        grid=(N // _ROWBLK,),
        in_specs=[
            pl.BlockSpec((_NC, _ROWBLK, D), lambda i: (0, i, 0)),
            pl.BlockSpec((_ROWBLK, D), lambda i: (i, 0)),
            pl.BlockSpec((3 * D, D), lambda i: (0, 0)),
            pl.BlockSpec((1, 3 * D), lambda i: (0, 0)),
            pl.BlockSpec((3 * D, D), lambda i: (0, 0)),
            pl.BlockSpec((1, 3 * D), lambda i: (0, 0)),
            pl.BlockSpec((D, D), lambda i: (0, 0)),
        ],
        out_specs=[
            pl.BlockSpec((_ROWBLK, D), lambda i: (i, 0)),
            pl.BlockSpec((_ROWBLK, D), lambda i: (i, 0)),
        ],
        out_shape=[
            jax.ShapeDtypeStruct((N, D), jnp.float32),
            jax.ShapeDtypeStruct((N, D), jnp.float32),
        ],
    )(p, h, w_ih, b_ih, w_hh, b_hh, wnext)


def _tc_gate_fin(p, h, w_ih, b_ih, w_hh, b_hh, lin_w, lin_b):
    def body(p_ref, h_ref, wih_ref, bih_ref, whh_ref, bhh_ref, w_ref, b_ref,
             o_ref):
        hn = _gru(p_ref, h_ref, wih_ref, bih_ref, whh_ref, bhh_ref)
        o_ref[...] = lax.dot_general(
            jnp.maximum(hn, 0.0).astype(jnp.bfloat16), w_ref[...],
            (((1,), (1,)), ((), ())),
            preferred_element_type=jnp.float32) + b_ref[...]

    return pl.pallas_call(
        body,
        grid=(N // _ROWBLK,),
        in_specs=[
            pl.BlockSpec((_NC, _ROWBLK, D), lambda i: (0, i, 0)),
            pl.BlockSpec((_ROWBLK, D), lambda i: (i, 0)),
            pl.BlockSpec((3 * D, D), lambda i: (0, 0)),
            pl.BlockSpec((1, 3 * D), lambda i: (0, 0)),
            pl.BlockSpec((3 * D, D), lambda i: (0, 0)),
            pl.BlockSpec((1, 3 * D), lambda i: (0, 0)),
            pl.BlockSpec((D, D), lambda i: (0, 0)),
            pl.BlockSpec((1, D), lambda i: (0, 0)),
        ],
        out_specs=pl.BlockSpec((_ROWBLK, D), lambda i: (i, 0)),
        out_shape=jax.ShapeDtypeStruct((N, D), jnp.float32),
    )(p, h, w_ih, b_ih, w_hh, b_hh, lin_w, lin_b)


def kernel(x, edge_index, edge_attr, weight, W_ih, W_hh, b_ih, b_hh, emb, lin_W, lin_b):
    src2d = edge_index[0].reshape(E // _CH, _CH)
    dst2d = edge_index[1].reshape(E // _CH, _CH)
    zeros = jnp.zeros((_ZR, D), jnp.float32)
    b_ih2 = b_ih.reshape(1, 3 * D)
    b_hh2 = b_hh.reshape(1, 3 * D)
    lin_b2 = lin_b.reshape(1, D)
    wb = weight.astype(jnp.bfloat16)
    wih_b = W_ih.astype(jnp.bfloat16)
    whh_b = W_hh.astype(jnp.bfloat16)
    lin_wb = lin_W.astype(jnp.bfloat16)

    h = x
    m = _tc_m(h, wb[0])
    for l in range(L - 1):
        p = _sc_agg(m, src2d, dst2d, zeros)
        h, m = _tc_gate(p, h, wih_b, b_ih2, whh_b, b_hh2, wb[l + 1])
    p = _sc_agg(m, src2d, dst2d, zeros)
    return _tc_gate_fin(p, h, wih_b, b_ih2, whh_b, b_hh2, lin_wb, lin_b2)


# speedup vs baseline: 12.4938x; 1.0088x over previous
import functools

import jax
import jax.numpy as jnp
from jax import lax
from jax.experimental import pallas as pl
from jax.experimental.pallas import tpu as pltpu
from jax.experimental.pallas import tpu_sc as plsc

N = 10000
E = 320000
D = 128
L = 3

_NC = 2
_NS = 16
_CH = 125
_TILE_E = E // (_NC * _NS)
_ITERS = _TILE_E // _CH
_ZR = 640
_NPAD = _ZR * _NS
_SB = 8
_NSB = _ITERS // _SB

_ROWBLK = 5000


def _sc_agg(m, src2d, dst2d, zeros):
    mesh = plsc.VectorSubcoreMesh(core_axis_name="c", subcore_axis_name="s")

    @functools.partial(
        pl.kernel,
        out_type=jax.ShapeDtypeStruct((_NC, N, D), jnp.float32),
        mesh=mesh,
        scratch_types=[
            pltpu.VMEM((_SB, _CH), jnp.int32),
            pltpu.VMEM((_SB, _CH), jnp.int32),
            pltpu.VMEM((_SB, _CH), jnp.int32),
            pltpu.VMEM((_SB, _CH), jnp.int32),
            pltpu.VMEM((_CH, D), jnp.float32),
            pltpu.VMEM((_CH, D), jnp.float32),
            pltpu.VMEM_SHARED((_NPAD, D), jnp.float32),
            pltpu.SemaphoreType.DMA,
            pltpu.SemaphoreType.DMA,
            pltpu.SemaphoreType.DMA,
            pltpu.SemaphoreType.DMA,
            pltpu.SemaphoreType.DMA,
            pltpu.SemaphoreType.DMA,
            pltpu.SemaphoreType.DMA,
        ],
    )
    def k(m_hbm, src_hbm, dst_hbm, z_hbm, out_hbm,
          si0, si1, di0, di1, r0, r1, agg_sh, is0, is1, gs0, gs1, ss0, ss1,
          zsem):
        sidx = (si0, si1)
        didx = (di0, di1)
        data = (r0, r1)
        isem = (is0, is1)
        gsem = (gs0, gs1)
        ssem = (ss0, ss1)
        c = lax.axis_index("c")
        s = lax.axis_index("s")
        zcp = pltpu.make_async_copy(z_hbm, agg_sh.at[pl.ds(s * _ZR, _ZR)], zsem)
        zcp.start()
        q0 = (c * _NS + s) * _ITERS

        def idx_load(u, p):
            q = q0 + u * _SB
            pltpu.make_async_copy(src_hbm.at[pl.ds(q, _SB)], sidx[p], isem[p]).start()
            pltpu.make_async_copy(dst_hbm.at[pl.ds(q, _SB)], didx[p], isem[p]).start()

        def idx_wait(p):
            pltpu.make_async_copy(src_hbm.at[pl.ds(q0, _SB)], sidx[p], isem[p]).wait()
            pltpu.make_async_copy(dst_hbm.at[pl.ds(q0, _SB)], didx[p], isem[p]).wait()

        def gather_start(p, j, b):
            pltpu.make_async_copy(m_hbm.at[sidx[p].at[j]], data[b], gsem[b]).start()

        def gather_wait(p, j, b):
            pltpu.make_async_copy(m_hbm.at[sidx[p].at[j]], data[b], gsem[b]).wait()

        def scatter_start(p, j, b):
            pltpu.async_copy(data[b], agg_sh.at[didx[p].at[j]], ssem[b], add=True)

        def scatter_wait(p, j, b):
            pltpu.make_async_copy(data[b], agg_sh.at[didx[p].at[j]], ssem[b]).wait()

        def super_visit(u, p, load_next2, next_block, first=False):
            for j in range(_SB):
                b = j % 2
                nb = (j + 1) % 2
                if j < _SB - 1:
                    if not (first and j == 0):
                        scatter_wait(p, j, nb)
                    gather_start(p, j + 1, nb)
                elif next_block:
                    scatter_wait(p, j, nb)
                    idx_wait(1 - p)
                    gather_start(1 - p, 0, 0)
                gather_wait(p, j, b)
                scatter_start(p, j, b)
                if j == _SB - 1 and load_next2:
                    idx_load(u + 2, p)

        idx_load(0, 0)
        idx_load(1, 1)
        idx_wait(0)
        gather_start(0, 0, 0)
        zcp.wait()
        plsc.subcore_barrier()

        super_visit(0, 0, True, True, first=True)
        super_visit(1, 1, True, True)

        @pl.loop(0, (_NSB - 4) // 2)
        def _(i):
            super_visit(2 * i + 2, 0, True, True)
            super_visit(2 * i + 3, 1, True, True)

        super_visit(_NSB - 2, 0, False, True)
        super_visit(_NSB - 1, 1, False, False)
        scatter_wait(1, _SB - 2, 0)
        scatter_wait(1, _SB - 1, 1)

        plsc.subcore_barrier()
        ob = lax.min(s * _ZR, N - _ZR)
        pltpu.sync_copy(agg_sh.at[pl.ds(ob, _ZR)],
                        out_hbm.at[c].at[pl.ds(ob, _ZR)])

    return k(m, src2d, dst2d, zeros)


def _tc_m(h, wl):
    def body(h_ref, wl_ref, m_ref):
        m_ref[...] = jnp.dot(h_ref[...].astype(jnp.bfloat16), wl_ref[...],
                             preferred_element_type=jnp.float32)

    return pl.pallas_call(
        body,
        grid=(N // _ROWBLK,),
        in_specs=[
            pl.BlockSpec((_ROWBLK, D), lambda i: (i, 0)),
            pl.BlockSpec((D, D), lambda i: (0, 0)),
        ],
        out_specs=pl.BlockSpec((_ROWBLK, D), lambda i: (i, 0)),
        out_shape=jax.ShapeDtypeStruct((N, D), jnp.float32),
    )(h, wl)


def _gru(p_ref, h_ref, wih_ref, bih_ref, whh_ref, bhh_ref):
    agg = (p_ref[0] + p_ref[1]).astype(jnp.bfloat16)
    gi = lax.dot_general(
        agg, wih_ref[...], (((1,), (1,)), ((), ())),
        preferred_element_type=jnp.float32) + bih_ref[...]
    gh = lax.dot_general(
        h_ref[...].astype(jnp.bfloat16), whh_ref[...], (((1,), (1,)), ((), ())),
        preferred_element_type=jnp.float32) + bhh_ref[...]
    r = jax.nn.sigmoid(gi[:, :D] + gh[:, :D])
    z = jax.nn.sigmoid(gi[:, D:2 * D] + gh[:, D:2 * D])
    n = jnp.tanh(gi[:, 2 * D:] + r * gh[:, 2 * D:])
    return (1.0 - z) * n + z * h_ref[...]


def _tc_gate(p, h, w_ih, b_ih, w_hh, b_hh, wnext):
    def body(p_ref, h_ref, wih_ref, bih_ref, whh_ref, bhh_ref, wn_ref,
             hn_ref, m_ref):
        hn = _gru(p_ref, h_ref, wih_ref, bih_ref, whh_ref, bhh_ref)
        hn_ref[...] = hn
        m_ref[...] = jnp.dot(hn.astype(jnp.bfloat16), wn_ref[...],
                             preferred_element_type=jnp.float32)

    return pl.pallas_call(
        body,
        grid=(N // _ROWBLK,),
        in_specs=[
            pl.BlockSpec((_NC, _ROWBLK, D), lambda i: (0, i, 0)),
            pl.BlockSpec((_ROWBLK, D), lambda i: (i, 0)),
            pl.BlockSpec((3 * D, D), lambda i: (0, 0)),
            pl.BlockSpec((1, 3 * D), lambda i: (0, 0)),
            pl.BlockSpec((3 * D, D), lambda i: (0, 0)),
            pl.BlockSpec((1, 3 * D), lambda i: (0, 0)),
            pl.BlockSpec((D, D), lambda i: (0, 0)),
        ],
        out_specs=[
            pl.BlockSpec((_ROWBLK, D), lambda i: (i, 0)),
            pl.BlockSpec((_ROWBLK, D), lambda i: (i, 0)),
        ],
        out_shape=[
            jax.ShapeDtypeStruct((N, D), jnp.float32),
            jax.ShapeDtypeStruct((N, D), jnp.float32),
        ],
    )(p, h, w_ih, b_ih, w_hh, b_hh, wnext)


def _tc_gate_fin(p, h, w_ih, b_ih, w_hh, b_hh, lin_w, lin_b):
    def body(p_ref, h_ref, wih_ref, bih_ref, whh_ref, bhh_ref, w_ref, b_ref,
             o_ref):
        hn = _gru(p_ref, h_ref, wih_ref, bih_ref, whh_ref, bhh_ref)
        o_ref[...] = lax.dot_general(
            jnp.maximum(hn, 0.0).astype(jnp.bfloat16), w_ref[...],
            (((1,), (1,)), ((), ())),
            preferred_element_type=jnp.float32) + b_ref[...]

    return pl.pallas_call(
        body,
        grid=(N // _ROWBLK,),
        in_specs=[
            pl.BlockSpec((_NC, _ROWBLK, D), lambda i: (0, i, 0)),
            pl.BlockSpec((_ROWBLK, D), lambda i: (i, 0)),
            pl.BlockSpec((3 * D, D), lambda i: (0, 0)),
            pl.BlockSpec((1, 3 * D), lambda i: (0, 0)),
            pl.BlockSpec((3 * D, D), lambda i: (0, 0)),
            pl.BlockSpec((1, 3 * D), lambda i: (0, 0)),
            pl.BlockSpec((D, D), lambda i: (0, 0)),
            pl.BlockSpec((1, D), lambda i: (0, 0)),
        ],
        out_specs=pl.BlockSpec((_ROWBLK, D), lambda i: (i, 0)),
        out_shape=jax.ShapeDtypeStruct((N, D), jnp.float32),
    )(p, h, w_ih, b_ih, w_hh, b_hh, lin_w, lin_b)


def kernel(x, edge_index, edge_attr, weight, W_ih, W_hh, b_ih, b_hh, emb, lin_W, lin_b):
    src2d = edge_index[0].reshape(E // _CH, _CH)
    dst2d = edge_index[1].reshape(E // _CH, _CH)
    zeros = jnp.zeros((_ZR, D), jnp.float32)
    b_ih2 = b_ih.reshape(1, 3 * D)
    b_hh2 = b_hh.reshape(1, 3 * D)
    lin_b2 = lin_b.reshape(1, D)
    wb = weight.astype(jnp.bfloat16)
    wih_b = W_ih.astype(jnp.bfloat16)
    whh_b = W_hh.astype(jnp.bfloat16)
    lin_wb = lin_W.astype(jnp.bfloat16)

    h = x
    m = _tc_m(h, wb[0])
    for l in range(L - 1):
        p = _sc_agg(m, src2d, dst2d, zeros)
        h, m = _tc_gate(p, h, wih_b, b_ih2, whh_b, b_hh2, wb[l + 1])
    p = _sc_agg(m, src2d, dst2d, zeros)
    return _tc_gate_fin(p, h, wih_b, b_ih2, whh_b, b_hh2, lin_wb, lin_b2)
